# baseline (speedup 1.0000x reference)
import functools
import math

import jax
import jax.numpy as jnp
from jax import lax
from jax.experimental import pallas as pl
from jax.experimental.pallas import tpu as pltpu
from jax.experimental.pallas import tpu_sc as plsc

D_MODEL = 128
LANES = 16
NUM_CORES = 2
NUM_SUBCORES = 16
NUM_WORKERS = NUM_CORES * NUM_SUBCORES
SCALE = math.sqrt(float(D_MODEL))


@functools.partial(jax.jit, static_argnames=("seq", "batch"))
def _run(x, table, pe2d, *, seq, batch):
    spw = seq // NUM_WORKERS
    rpw = spw * batch
    n_chunks = rpw // D_MODEL
    cs = spw // n_chunks
    crows = cs * batch

    mesh = plsc.VectorSubcoreMesh(
        core_axis_name="c", subcore_axis_name="s",
        num_cores=NUM_CORES, num_subcores=NUM_SUBCORES)

    @functools.partial(
        pl.kernel,
        out_type=jax.ShapeDtypeStruct((seq * batch, D_MODEL), jnp.float32),
        mesh=mesh,
        scratch_types=[
            pltpu.VMEM((spw, batch), jnp.int32),
            pltpu.VMEM((n_chunks, D_MODEL), jnp.int32),
            pltpu.VMEM((rpw, D_MODEL), jnp.float32),
            pltpu.VMEM((spw, D_MODEL), jnp.float32),
            [pltpu.SemaphoreType.DMA] * n_chunks,
            pltpu.SemaphoreType.DMA,
        ],
        compiler_params=pltpu.CompilerParams(needs_layout_passes=False),
    )
    def run(x_hbm, table_hbm, pe_hbm, out_hbm, stage_v, idx_v, rows_v, pe_v,
            gsems, st_sem):
        wid = lax.axis_index("s") * NUM_CORES + lax.axis_index("c")
        base = wid * spw

        pltpu.sync_copy(x_hbm.at[pl.ds(base, spw)], stage_v)
        lane = jnp.arange(LANES, dtype=jnp.int32)
        for k in range(rpw // LANES):
            p = k * LANES + lane
            vals = plsc.load_gather(stage_v, [p >> 2, p & 3])
            idx_v[k * LANES // D_MODEL,
                  pl.ds(k * LANES % D_MODEL, LANES)] = vals

        gathers = [
            pltpu.async_copy(
                table_hbm.at[idx_v.at[g]],
                rows_v.at[pl.ds(g * crows, crows)], gsems[g])
            for g in range(n_chunks)
        ]
        pltpu.sync_copy(pe_hbm.at[pl.ds(base, spw)], pe_v)

        stores = []
        for g in range(n_chunks):
            gathers[g].wait()

            @pl.loop(g * cs, (g + 1) * cs)
            def _(s):
                pv = [pe_v[s, pl.ds(j * LANES, LANES)]
                      for j in range(D_MODEL // LANES)]
                for b in range(batch):
                    r = s * batch + b
                    for j in range(D_MODEL // LANES):
                        sl = pl.ds(j * LANES, LANES)
                        rows_v[r, sl] = rows_v[r, sl] * SCALE + pv[j]

            stores.append(pltpu.async_copy(
                rows_v.at[pl.ds(g * crows, crows)],
                out_hbm.at[pl.ds(wid * rpw + g * crows, crows)], st_sem))
        for st in stores:
            st.wait()

    return run(x, table, pe2d)


def kernel(x, table, pe):
    seq, batch = x.shape
    pe2d = pe.reshape(pe.shape[0], D_MODEL)
    out = _run(x, table, pe2d, seq=seq, batch=batch)
    return out.reshape(seq, batch, D_MODEL)


# speedup vs baseline: 1.0281x; 1.0281x over previous
import functools
import math

import jax
import jax.numpy as jnp
from jax import lax
from jax.experimental import pallas as pl
from jax.experimental.pallas import tpu as pltpu
from jax.experimental.pallas import tpu_sc as plsc

D_MODEL = 128
LANES = 16
NUM_CORES = 2
NUM_SUBCORES = 16
NUM_WORKERS = NUM_CORES * NUM_SUBCORES
SCALE = math.sqrt(float(D_MODEL))


@functools.partial(jax.jit, static_argnames=("seq", "batch"))
def _run(x2d, table, pe2d, *, seq, batch):
    spw = seq // NUM_WORKERS
    rpw = spw * batch
    n_idx_rows = rpw // D_MODEL
    n_chunks = 4
    crows = rpw // n_chunks
    cs = spw // n_chunks

    mesh = plsc.VectorSubcoreMesh(
        core_axis_name="c", subcore_axis_name="s",
        num_cores=NUM_CORES, num_subcores=NUM_SUBCORES)

    @functools.partial(
        pl.kernel,
        out_type=jax.ShapeDtypeStruct((seq * batch, D_MODEL), jnp.float32),
        mesh=mesh,
        scratch_types=[
            pltpu.VMEM((n_idx_rows, D_MODEL), jnp.int32),
            pltpu.VMEM((rpw, D_MODEL), jnp.float32),
            pltpu.VMEM((spw, D_MODEL), jnp.float32),
            [pltpu.SemaphoreType.DMA] * n_chunks,
            pltpu.SemaphoreType.DMA,
            pltpu.SemaphoreType.DMA,
        ],
    )
    def run(x_hbm, table_hbm, pe_hbm, out_hbm, idx_v, rows_v, pe_v,
            gsems, st_sem, pe_sem):
        wid = lax.axis_index("s") * NUM_CORES + lax.axis_index("c")
        base = wid * spw

        pltpu.sync_copy(x_hbm.at[pl.ds(wid * n_idx_rows, n_idx_rows)], idx_v)
        pe_cp = pltpu.async_copy(pe_hbm.at[pl.ds(base, spw)], pe_v, pe_sem)
        gathers = [
            pltpu.async_copy(
                table_hbm.at[idx_v.at[c * crows // D_MODEL,
                                      pl.ds(c * crows % D_MODEL, crows)]],
                rows_v.at[pl.ds(c * crows, crows)], gsems[c])
            for c in range(n_chunks)
        ]
        pe_cp.wait()

        stores = []
        for c in range(n_chunks):
            gathers[c].wait()

            @pl.loop(c * cs, (c + 1) * cs)
            def _(s):
                pv = [pe_v[s, pl.ds(j * LANES, LANES)]
                      for j in range(D_MODEL // LANES)]
                for b in range(batch):
                    r = s * batch + b
                    for j in range(D_MODEL // LANES):
                        sl = pl.ds(j * LANES, LANES)
                        rows_v[r, sl] = rows_v[r, sl] * SCALE + pv[j]

            stores.append(pltpu.async_copy(
                rows_v.at[pl.ds(c * crows, crows)],
                out_hbm.at[pl.ds(wid * rpw + c * crows, crows)], st_sem))
        for st in stores:
            st.wait()

    return run(x2d, table, pe2d)


def kernel(x, table, pe):
    seq, batch = x.shape
    x2d = x.reshape(seq * batch // D_MODEL, D_MODEL)
    pe2d = pe.reshape(pe.shape[0], D_MODEL)
    out = _run(x2d, table, pe2d, seq=seq, batch=batch)
    return out.reshape(seq, batch, D_MODEL)
